# baseline (speedup 1.0000x reference)
import functools
import jax
import jax.numpy as jnp
from jax import lax
from jax.experimental import pallas as pl
from jax.experimental.pallas import tpu as pltpu
from jax.experimental.pallas import tpu_sc as plsc

B, L = 8, 448
SEQ_EMB = 32
RES_DIM = 128
PAIR_DIM = 64
MAX_REL = 32
NREL = 2 * MAX_REL + 1
NUM_EMB = 5
TW = 144

_HI = jax.lax.Precision.HIGHEST

NC, NS = 2, 16
NW = NC * NS
ROWS = B * L
RPW = ROWS // NW
CB = 8


def _prep_body(seq_ref, dih_ref, ent_ref, acc_ref, con_ref, emb_ref, pe_ref,
               rel_emb_ref, Wr_ref, br_ref, Wp_ref, bp_ref,
               res_out, e2t_out):
    seq = seq_ref[...]
    onehot = (seq[..., None] ==
              jax.lax.broadcasted_iota(jnp.int32, (B, L, NUM_EMB), 2)
              ).astype(jnp.float32)
    M = jax.lax.dot_general(emb_ref[...], Wr_ref[0:SEQ_EMB, :],
                            (((1,), (0,)), ((), ())), precision=_HI)
    res = jax.lax.dot_general(onehot.reshape(B * L, NUM_EMB), M,
                              (((1,), (0,)), ((), ())), precision=_HI)
    res = res + jax.lax.dot_general(
        dih_ref[...].reshape(B * L, 4), Wr_ref[SEQ_EMB:SEQ_EMB + 4, :],
        (((1,), (0,)), ((), ())), precision=_HI)
    res = res.reshape(B, L, RES_DIM)
    res = res + ent_ref[...][..., None] * Wr_ref[SEQ_EMB + 4, :][None, None, :]
    res = res + acc_ref[...][..., None] * Wr_ref[SEQ_EMB + 5, :][None, None, :]
    res = res + con_ref[...][..., None] * Wr_ref[SEQ_EMB + 6, :][None, None, :]
    res = res + br_ref[...][None, None, :]
    res = res + pe_ref[0, :L, :][None]
    res_out[...] = res

    relproj = jax.lax.dot_general(
        rel_emb_ref[...], Wp_ref[2:, :], (((1,), (0,)), ((), ())),
        precision=_HI) + bp_ref[...][None, :]
    kk = jax.lax.broadcasted_iota(jnp.int32, (NREL, TW), 0)
    tt = jax.lax.broadcasted_iota(jnp.int32, (NREL, TW), 1)
    main = (tt < 128) & (jnp.clip(tt - 32, 0, 2 * MAX_REL) == kk)
    relx = ((tt == 130) & (kk == 0)) | ((tt == 131) & (kk == 2 * MAX_REL))
    sel_r = (main | relx).astype(jnp.float32)
    kk2 = jax.lax.broadcasted_iota(jnp.int32, (2, TW), 0)
    tt2 = jax.lax.broadcasted_iota(jnp.int32, (2, TW), 1)
    sel_w = (((kk2 == 0) & (tt2 == 128)) |
             ((kk2 == 1) & (tt2 == 129))).astype(jnp.float32)
    e2t = jax.lax.dot_general(relproj, sel_r, (((0,), (0,)), ((), ())),
                              precision=_HI)
    e2t = e2t + jax.lax.dot_general(Wp_ref[0:2, :], sel_w,
                                    (((0,), (0,)), ((), ())), precision=_HI)
    e2t_out[...] = e2t


_GDN = lax.GatherDimensionNumbers(offset_dims=(), collapsed_slice_dims=(0,),
                                  start_index_map=(0,))


def _bcast(ch, u):
    return lax.gather(ch, jnp.full((16, 1), u, jnp.int32), _GDN, (1,),
                      mode=lax.GatherScatterMode.PROMISE_IN_BOUNDS)


def _perm(ch, idxv):
    return lax.gather(ch, idxv[:, None], _GDN, (1,),
                      mode=lax.GatherScatterMode.PROMISE_IN_BOUNDS)


NSLOT = 2
HCH = PAIR_DIM // 2


def _sc_pair_body(e2t_hbm, p_hbm, c_hbm, out_hbm,
                  e2t, pv, cv, ov, *sems):
    psems = sems[0:NSLOT]
    csems = sems[NSLOT:2 * NSLOT]
    osems = sems[2 * NSLOT:4 * NSLOT]
    wid = lax.axis_index("s") * NC + lax.axis_index("c")
    base_row = wid * RPW
    pltpu.sync_copy(e2t_hbm, e2t)
    lane = lax.iota(jnp.int32, 16)

    def row_bi(r):
        row = base_row + r
        b = row // L
        return b, row - b * L

    def ohalf(ss, h, b, i, sem):
        return pltpu.make_async_copy(
            ov.at[pl.ds(ss * PAIR_DIM + h * HCH, HCH)],
            out_hbm.at[b, i, pl.ds(h * HCH, HCH)], sem)

    for ss in range(NSLOT):
        b, i = row_bi(ss)
        pltpu.make_async_copy(p_hbm.at[b, i], pv.at[ss], psems[ss]).start()
        pltpu.make_async_copy(c_hbm.at[b, i], cv.at[ss], csems[ss]).start()

    def row_body(r, _):
        slot = lax.rem(r, NSLOT)
        obase = slot * PAIR_DIM
        b, i = row_bi(r)
        for ss in range(NSLOT):
            @pl.when(slot == ss)
            def _(ss=ss):
                pltpu.make_async_copy(p_hbm.at[b, i], pv.at[ss],
                                      psems[ss]).wait()
                pltpu.make_async_copy(c_hbm.at[b, i], cv.at[ss],
                                      csems[ss]).wait()
                @pl.when(r >= NSLOT)
                def _():
                    pb_, pi_ = row_bi(r - NSLOT)
                    ohalf(ss, 0, pb_, pi_, osems[2 * ss]).wait()
                    ohalf(ss, 1, pb_, pi_, osems[2 * ss + 1]).wait()

        glo = jnp.maximum((i - (MAX_REL - 1)) // 16, 0)
        ghi = jnp.minimum((i + (MAX_REL - 1)) // 16, L // 16 - 1)
        woff = 64 - i
        rot = woff % 16
        ashift = woff - rot
        idxv = (lane + rot) % 16
        lmask = lane < (16 - rot)

        for cb in range(PAIR_DIM // CB):
            aux = [None] * CB
            w0s = [None] * CB
            w1s = [None] * CB
            rel0s = [None] * CB
            rel64s = [None] * CB
            for cc in range(CB):
                ch = cb * CB + cc
                aux[cc] = e2t[ch, pl.ds(128, 16)]
                w0s[cc] = _bcast(aux[cc], 0)
                w1s[cc] = _bcast(aux[cc], 1)
                rel0s[cc] = _bcast(aux[cc], 2)
                rel64s[cc] = _bcast(aux[cc], 3)

            def mk_const(rels):
                def body(jg):
                    jbase = pl.multiple_of(jg * 16, 16)
                    pch = pv[slot, pl.ds(jbase, 16)]
                    cch = cv[slot, pl.ds(jbase, 16)]
                    for cc in range(CB):
                        ch = cb * CB + cc
                        ov[obase + ch, pl.ds(jbase, 16)] = (
                            pch * w0s[cc] + cch * w1s[cc] + rels[cc])
                return body

            def band_body(jg):
                jbase = pl.multiple_of(jg * 16, 16)
                pch = pv[slot, pl.ds(jbase, 16)]
                cch = cv[slot, pl.ds(jbase, 16)]
                a = pl.multiple_of(jbase + ashift, 16)
                for cc in range(CB):
                    ch = cb * CB + cc
                    c0 = e2t[ch, pl.ds(a, 16)]
                    c1 = e2t[ch, pl.ds(a + 16, 16)]
                    relt = jnp.where(lmask, _perm(c0, idxv),
                                     _perm(c1, idxv))
                    ov[obase + ch, pl.ds(jbase, 16)] = (
                        pch * w0s[cc] + cch * w1s[cc] + relt)

            plsc.parallel_loop(0, glo, unroll=2)(mk_const(rel0s))
            plsc.parallel_loop(glo, ghi + 1)(band_body)
            plsc.parallel_loop(ghi + 1, L // 16, unroll=2)(mk_const(rel64s))

            if cb == (PAIR_DIM // CB) // 2 - 1:
                for ss in range(NSLOT):
                    @pl.when(slot == ss)
                    def _(ss=ss):
                        ohalf(ss, 0, b, i, osems[2 * ss]).start()

        for ss in range(NSLOT):
            @pl.when(slot == ss)
            def _(ss=ss):
                ohalf(ss, 1, b, i, osems[2 * ss + 1]).start()

                @pl.when(r + NSLOT < RPW)
                def _():
                    nb, ni = row_bi(r + NSLOT)
                    pltpu.make_async_copy(p_hbm.at[nb, ni], pv.at[ss],
                                          psems[ss]).start()
                    pltpu.make_async_copy(c_hbm.at[nb, ni], cv.at[ss],
                                          csems[ss]).start()
        return 0

    lax.fori_loop(0, RPW, row_body, 0, unroll=False)
    for ss in range(NSLOT):
        b, i = row_bi(RPW - NSLOT + ss)
        ohalf(ss, 0, b, i, osems[2 * ss]).wait()
        ohalf(ss, 1, b, i, osems[2 * ss + 1]).wait()


@functools.lru_cache(maxsize=1)
def _sc_pair():
  return pl.kernel(
    _sc_pair_body,
    out_type=jax.ShapeDtypeStruct((B, L, PAIR_DIM, L), jnp.float32),
    mesh=plsc.VectorSubcoreMesh(core_axis_name="c", subcore_axis_name="s",
                                num_cores=NC, num_subcores=NS),
    scratch_types=(
        [
            pltpu.VMEM((PAIR_DIM, TW), jnp.float32),
            pltpu.VMEM((NSLOT, L), jnp.float32),
            pltpu.VMEM((NSLOT, L), jnp.float32),
            pltpu.VMEM((NSLOT * PAIR_DIM, L), jnp.float32)
        ] + [pltpu.SemaphoreType.DMA] * (4 * NSLOT)
    ),
  )


@jax.jit
def _impl(sequence_int, dihedral_features, pairing_probs, positional_entropy,
          coupling_matrix, accessibility, conservation, emb_table, pe,
          rel_emb, W_res, b_res, W_pair, b_pair):
    res, e2t = pl.pallas_call(
        _prep_body,
        out_shape=(
            jax.ShapeDtypeStruct((B, L, RES_DIM), jnp.float32),
            jax.ShapeDtypeStruct((PAIR_DIM, TW), jnp.float32),
        ),
    )(sequence_int.astype(jnp.int32), dihedral_features, positional_entropy,
      accessibility, conservation, emb_table, pe, rel_emb, W_res, b_res,
      W_pair, b_pair)

    pair_t = _sc_pair()(e2t, pairing_probs, coupling_matrix)
    return res, jnp.swapaxes(pair_t, 2, 3)


def kernel(sequence_int, mask, dihedral_features, pairing_probs,
           positional_entropy, coupling_matrix, accessibility, conservation,
           emb_table, pe, rel_emb, W_res, b_res, W_pair, b_pair):
    res, pair = _impl(sequence_int, dihedral_features, pairing_probs,
                      positional_entropy, coupling_matrix, accessibility,
                      conservation, emb_table, pe, rel_emb, W_res, b_res,
                      W_pair, b_pair)
    return res, pair, mask


# speedup vs baseline: 1.2586x; 1.2586x over previous
import functools
import jax
import jax.numpy as jnp
from jax import lax
from jax.experimental import pallas as pl
from jax.experimental.pallas import tpu as pltpu
from jax.experimental.pallas import tpu_sc as plsc

B, L = 8, 448
SEQ_EMB = 32
RES_DIM = 128
PAIR_DIM = 64
MAX_REL = 32
NREL = 2 * MAX_REL + 1
NUM_EMB = 5
TW = 144

_HI = jax.lax.Precision.HIGHEST

NC, NS = 2, 16
NW = NC * NS
ROWS = B * L
RPW = ROWS // NW
CB = 8


def _prep_body(seq_ref, dih_ref, ent_ref, acc_ref, con_ref, emb_ref, pe_ref,
               rel_emb_ref, Wr_ref, br_ref, Wp_ref, bp_ref,
               res_out, e2t_out):
    seq = seq_ref[...]
    onehot = (seq[..., None] ==
              jax.lax.broadcasted_iota(jnp.int32, (B, L, NUM_EMB), 2)
              ).astype(jnp.float32)
    M = jax.lax.dot_general(emb_ref[...], Wr_ref[0:SEQ_EMB, :],
                            (((1,), (0,)), ((), ())), precision=_HI)
    res = jax.lax.dot_general(onehot.reshape(B * L, NUM_EMB), M,
                              (((1,), (0,)), ((), ())), precision=_HI)
    res = res + jax.lax.dot_general(
        dih_ref[...].reshape(B * L, 4), Wr_ref[SEQ_EMB:SEQ_EMB + 4, :],
        (((1,), (0,)), ((), ())), precision=_HI)
    res = res.reshape(B, L, RES_DIM)
    res = res + ent_ref[...][..., None] * Wr_ref[SEQ_EMB + 4, :][None, None, :]
    res = res + acc_ref[...][..., None] * Wr_ref[SEQ_EMB + 5, :][None, None, :]
    res = res + con_ref[...][..., None] * Wr_ref[SEQ_EMB + 6, :][None, None, :]
    res = res + br_ref[...][None, None, :]
    res = res + pe_ref[0, :L, :][None]
    res_out[...] = res

    relproj = jax.lax.dot_general(
        rel_emb_ref[...], Wp_ref[2:, :], (((1,), (0,)), ((), ())),
        precision=_HI) + bp_ref[...][None, :]
    kk = jax.lax.broadcasted_iota(jnp.int32, (NREL, TW), 0)
    tt = jax.lax.broadcasted_iota(jnp.int32, (NREL, TW), 1)
    main = (tt < 128) & (jnp.clip(tt - 32, 0, 2 * MAX_REL) == kk)
    relx = ((tt == 130) & (kk == 0)) | ((tt == 131) & (kk == 2 * MAX_REL))
    sel_r = (main | relx).astype(jnp.float32)
    kk2 = jax.lax.broadcasted_iota(jnp.int32, (2, TW), 0)
    tt2 = jax.lax.broadcasted_iota(jnp.int32, (2, TW), 1)
    sel_w = (((kk2 == 0) & (tt2 == 128)) |
             ((kk2 == 1) & (tt2 == 129))).astype(jnp.float32)
    e2t = jax.lax.dot_general(relproj, sel_r, (((0,), (0,)), ((), ())),
                              precision=_HI)
    e2t = e2t + jax.lax.dot_general(Wp_ref[0:2, :], sel_w,
                                    (((0,), (0,)), ((), ())), precision=_HI)
    e2t_out[...] = e2t


_GDN = lax.GatherDimensionNumbers(offset_dims=(), collapsed_slice_dims=(0,),
                                  start_index_map=(0,))


def _bcast(ch, u):
    return lax.gather(ch, jnp.full((16, 1), u, jnp.int32), _GDN, (1,),
                      mode=lax.GatherScatterMode.PROMISE_IN_BOUNDS)


def _perm(ch, idxv):
    return lax.gather(ch, idxv[:, None], _GDN, (1,),
                      mode=lax.GatherScatterMode.PROMISE_IN_BOUNDS)


NSLOT = 2
HCH = PAIR_DIM // 2


def _sc_pair_body(e2t_hbm, p_hbm, c_hbm, out_hbm,
                  e2t, pv, cv, ov, *sems):
    psems = sems[0:NSLOT]
    csems = sems[NSLOT:2 * NSLOT]
    osems = sems[2 * NSLOT:4 * NSLOT]
    wid = lax.axis_index("s") * NC + lax.axis_index("c")
    base_row = wid * RPW
    pltpu.sync_copy(e2t_hbm, e2t)
    lane = lax.iota(jnp.int32, 16)

    def row_bi(r):
        row = base_row + r
        b = row // L
        return b, row - b * L

    def ohalf(ss, h, b, i, sem):
        return pltpu.make_async_copy(
            ov.at[pl.ds(ss * PAIR_DIM + h * HCH, HCH)],
            out_hbm.at[b, i, pl.ds(h * HCH, HCH)], sem)

    for ss in range(NSLOT):
        b, i = row_bi(ss)
        pltpu.make_async_copy(p_hbm.at[b, i], pv.at[ss], psems[ss]).start()
        pltpu.make_async_copy(c_hbm.at[b, i], cv.at[ss], csems[ss]).start()

    def adv(bb, ii):
        w = ii == L - 1
        return jnp.where(w, bb + 1, bb), jnp.where(w, 0, ii + 1)

    def row_body(r, carry):
        b, i, lb, li, nb, ni = carry
        slot = lax.rem(r, NSLOT)
        obase = slot * PAIR_DIM
        for ss in range(NSLOT):
            @pl.when(slot == ss)
            def _(ss=ss):
                pltpu.make_async_copy(p_hbm.at[b, i], pv.at[ss],
                                      psems[ss]).wait()
                pltpu.make_async_copy(c_hbm.at[b, i], cv.at[ss],
                                      csems[ss]).wait()
                @pl.when(r >= NSLOT)
                def _():
                    ohalf(ss, 0, lb, li, osems[2 * ss]).wait()
                    ohalf(ss, 1, lb, li, osems[2 * ss + 1]).wait()

        glo = jnp.maximum((i - (MAX_REL - 1)) // 16, 0)
        ghi = jnp.minimum((i + (MAX_REL - 1)) // 16, L // 16 - 1)
        woff = 64 - i
        rot = woff % 16
        ashift = woff - rot
        idxv = (lane + rot) % 16
        lmask = lane < (16 - rot)

        for cb in range(PAIR_DIM // CB):
            aux = [None] * CB
            w0s = [None] * CB
            w1s = [None] * CB
            rel0s = [None] * CB
            rel64s = [None] * CB
            for cc in range(CB):
                ch = cb * CB + cc
                aux[cc] = e2t[ch, pl.ds(128, 16)]
                w0s[cc] = _bcast(aux[cc], 0)
                w1s[cc] = _bcast(aux[cc], 1)
                rel0s[cc] = _bcast(aux[cc], 2)
                rel64s[cc] = _bcast(aux[cc], 3)

            def mk_const(rels):
                def body(jg):
                    jbase = pl.multiple_of(jg * 16, 16)
                    pch = pv[slot, pl.ds(jbase, 16)]
                    cch = cv[slot, pl.ds(jbase, 16)]
                    for cc in range(CB):
                        ch = cb * CB + cc
                        ov[obase + ch, pl.ds(jbase, 16)] = (
                            pch * w0s[cc] + cch * w1s[cc] + rels[cc])
                return body

            def band_body(jg):
                jbase = pl.multiple_of(jg * 16, 16)
                pch = pv[slot, pl.ds(jbase, 16)]
                cch = cv[slot, pl.ds(jbase, 16)]
                a = pl.multiple_of(jbase + ashift, 16)
                for cc in range(CB):
                    ch = cb * CB + cc
                    c0 = e2t[ch, pl.ds(a, 16)]
                    c1 = e2t[ch, pl.ds(a + 16, 16)]
                    relt = jnp.where(lmask, _perm(c0, idxv),
                                     _perm(c1, idxv))
                    ov[obase + ch, pl.ds(jbase, 16)] = (
                        pch * w0s[cc] + cch * w1s[cc] + relt)

            plsc.parallel_loop(0, glo)(mk_const(rel0s))
            plsc.parallel_loop(glo, ghi + 1)(band_body)
            plsc.parallel_loop(ghi + 1, L // 16)(mk_const(rel64s))

            if cb == (PAIR_DIM // CB) // 2 - 1:
                for ss in range(NSLOT):
                    @pl.when(slot == ss)
                    def _(ss=ss):
                        ohalf(ss, 0, b, i, osems[2 * ss]).start()

        for ss in range(NSLOT):
            @pl.when(slot == ss)
            def _(ss=ss):
                ohalf(ss, 1, b, i, osems[2 * ss + 1]).start()

                @pl.when(r + NSLOT < RPW)
                def _():
                    pltpu.make_async_copy(p_hbm.at[nb, ni], pv.at[ss],
                                          psems[ss]).start()
                    pltpu.make_async_copy(c_hbm.at[nb, ni], cv.at[ss],
                                          csems[ss]).start()
        return (*adv(b, i), *adv(lb, li), *adv(nb, ni))

    lax.fori_loop(0, RPW, row_body,
                  (*row_bi(0), *row_bi(-NSLOT), *row_bi(NSLOT)),
                  unroll=False)
    for ss in range(NSLOT):
        b, i = row_bi(RPW - NSLOT + ss)
        ohalf(ss, 0, b, i, osems[2 * ss]).wait()
        ohalf(ss, 1, b, i, osems[2 * ss + 1]).wait()


@functools.lru_cache(maxsize=1)
def _sc_pair():
  return pl.kernel(
    _sc_pair_body,
    out_type=jax.ShapeDtypeStruct((B, L, PAIR_DIM, L), jnp.float32),
    mesh=plsc.VectorSubcoreMesh(core_axis_name="c", subcore_axis_name="s",
                                num_cores=NC, num_subcores=NS),
    scratch_types=(
        [
            pltpu.VMEM((PAIR_DIM, TW), jnp.float32),
            pltpu.VMEM((NSLOT, L), jnp.float32),
            pltpu.VMEM((NSLOT, L), jnp.float32),
            pltpu.VMEM((NSLOT * PAIR_DIM, L), jnp.float32)
        ] + [pltpu.SemaphoreType.DMA] * (4 * NSLOT)
    ),
  )


@jax.jit
def _impl(sequence_int, dihedral_features, pairing_probs, positional_entropy,
          coupling_matrix, accessibility, conservation, emb_table, pe,
          rel_emb, W_res, b_res, W_pair, b_pair):
    res, e2t = pl.pallas_call(
        _prep_body,
        out_shape=(
            jax.ShapeDtypeStruct((B, L, RES_DIM), jnp.float32),
            jax.ShapeDtypeStruct((PAIR_DIM, TW), jnp.float32),
        ),
    )(sequence_int.astype(jnp.int32), dihedral_features, positional_entropy,
      accessibility, conservation, emb_table, pe, rel_emb, W_res, b_res,
      W_pair, b_pair)

    pair_t = _sc_pair()(e2t, pairing_probs, coupling_matrix)
    return res, jnp.swapaxes(pair_t, 2, 3)


def kernel(sequence_int, mask, dihedral_features, pairing_probs,
           positional_entropy, coupling_matrix, accessibility, conservation,
           emb_table, pe, rel_emb, W_res, b_res, W_pair, b_pair):
    res, pair = _impl(sequence_int, dihedral_features, pairing_probs,
                      positional_entropy, coupling_matrix, accessibility,
                      conservation, emb_table, pe, rel_emb, W_res, b_res,
                      W_pair, b_pair)
    return res, pair, mask
